# baseline (speedup 1.0000x reference)
import functools

import jax
import jax.numpy as jnp
from jax import lax
from jax.experimental import pallas as pl
from jax.experimental.pallas import tpu as pltpu
from jax.experimental.pallas import tpu_sc as plsc

BINS = 100000
EMB = 16
NF = 26

NC = 2
NS = 16
LANES = 16
NW = NC * NS

CHUNK = 128
ROWS_PER_SUPER = 8


def _sc_gather_call(tot):
    bpw = tot // NW
    nchunk = NF
    bt = bpw // NF

    mesh = plsc.VectorSubcoreMesh(core_axis_name="c", subcore_axis_name="s")

    @functools.partial(
        pl.kernel,
        out_type=jax.ShapeDtypeStruct((tot * EMB,), jnp.float32),
        mesh=mesh,
        scratch_types=[
            pltpu.VMEM((bpw,), jnp.int32),
            pltpu.VMEM((bpw,), jnp.int32),
            pltpu.VMEM((bpw,), jnp.int32),
            pltpu.VMEM((CHUNK, 128), jnp.float32),
            pltpu.VMEM((CHUNK, 128), jnp.float32),
            pltpu.VMEM((bpw * EMB,), jnp.float32),
            pltpu.SemaphoreType.DMA,
            pltpu.SemaphoreType.DMA,
        ],
        compiler_params=pltpu.CompilerParams(needs_layout_passes=False),
    )
    def sc_gather(idx_hbm, tbl_hbm, out_hbm, idx_v, sidx_v, soff_v, buf_a,
                  buf_b, rows_v, sem_a, sem_b):
        wid = lax.axis_index("s") * NC + lax.axis_index("c")
        base = wid * bpw
        pltpu.sync_copy(idx_hbm.at[pl.ds(base, bpw)], idx_v)

        lane = lax.broadcasted_iota(jnp.int32, (LANES,), 0)

        def xform(f, carry):
            for q in range(bt // LANES):
                bl = q * LANES + lane
                raw = plsc.load_gather(idx_v, [bl * NF + f])
                binned = lax.rem(raw, BINS)
                dst = pl.ds(f * CHUNK + q * LANES, LANES)
                sidx_v[dst] = lax.shift_right_logical(binned, 3)
                soff_v[dst] = (binned & 7) * EMB
            return carry

        lax.fori_loop(0, NF, xform, 0)

        bufs = (buf_a, buf_b)
        sems = (sem_a, sem_b)

        def fire(f, buf, sem):
            pltpu.make_async_copy(
                tbl_hbm.at[f].at[sidx_v.at[pl.ds(f * CHUNK, CHUNK)]], buf, sem,
            ).start()

        def extract(f, buf):
            for q in range(bt // LANES):
                bl = q * LANES + lane
                rowv = q * LANES + lane
                colb = soff_v[pl.ds(f * CHUNK + q * LANES, LANES)]
                posb = bl * (NF * EMB) + f * EMB
                for p in range(EMB):
                    val = plsc.load_gather(buf, [rowv, colb + p])
                    plsc.store_scatter(rows_v, [posb + p], val)

        fire(0, bufs[0], sems[0])

        def pairbody(g, carry):
            for b in range(2):
                f = g * 2 + b

                @pl.when(f + 1 < nchunk)
                def _():
                    fire(f + 1, bufs[1 - b], sems[1 - b])

                pltpu.make_async_copy(
                    tbl_hbm.at[0].at[pl.ds(0, CHUNK)], bufs[b], sems[b]).wait()
                extract(f, bufs[b])
            return carry

        lax.fori_loop(0, nchunk // 2, pairbody, 0)
        pltpu.sync_copy(rows_v, out_hbm.at[pl.ds(base * EMB, bpw * EMB)])

    return sc_gather


def _mlp_body(dense_ref, embs_ref, w1a, w1b, b1, w2, b2, w3, b3, w4, b4, out_ref):
    f32 = jnp.float32
    h = jnp.dot(embs_ref[...], w1b[...], preferred_element_type=f32)
    h += jnp.dot(dense_ref[...], w1a[...], preferred_element_type=f32)
    h = jnp.maximum(h + b1[...], 0.0)
    h = jnp.maximum(jnp.dot(h, w2[...], preferred_element_type=f32) + b2[...], 0.0)
    h = jnp.maximum(jnp.dot(h, w3[...], preferred_element_type=f32) + b3[...], 0.0)
    o = jnp.dot(h, w4[...], preferred_element_type=f32) + b4[...]
    out_ref[...] = 1.0 / (1.0 + jnp.exp(-o))


def _mlp_call(dense, embs, w1a, w1b, b1, w2, b2, w3, b3, w4, b4, bt=512):
    bsz, nd = dense.shape
    demb = embs.shape[1]
    u1, u2, u3 = w2.shape[0], w3.shape[0], w4.shape[0]
    grid = (bsz // bt,)
    full = lambda shape: pl.BlockSpec(shape, lambda i: (0, 0))
    return pl.pallas_call(
        _mlp_body,
        grid=grid,
        in_specs=[
            pl.BlockSpec((bt, nd), lambda i: (i, 0)),
            pl.BlockSpec((bt, demb), lambda i: (i, 0)),
            full((nd, u1)),
            full((demb, u1)),
            full((1, u1)),
            full((u1, u2)),
            full((1, u2)),
            full((u2, u3)),
            full((1, u3)),
            full((u3, 1)),
            full((1, 1)),
        ],
        out_specs=pl.BlockSpec((bt, 1), lambda i: (i, 0)),
        out_shape=jax.ShapeDtypeStruct((bsz, 1), jnp.float32),
    )(dense, embs, w1a, w1b, b1, w2, b2, w3, b3, w4, b4)


def kernel(dense, sparse_idx, emb_table, W1, b1, W2, b2, W3, b3, W4, b4):
    bsz, nd = dense.shape
    nf, nbins, emb = emb_table.shape
    tot = bsz * nf

    idx_flat = sparse_idx.reshape(tot).astype(jnp.int32)
    tbl8 = emb_table.reshape(nf, nbins // ROWS_PER_SUPER, ROWS_PER_SUPER * emb)
    rows = _sc_gather_call(tot)(idx_flat, tbl8)
    embs = rows.reshape(bsz, nf * emb)
    return _mlp_call(
        dense, embs,
        W1[:nd], W1[nd:], b1.reshape(1, -1),
        W2, b2.reshape(1, -1),
        W3, b3.reshape(1, -1),
        W4, b4.reshape(1, -1),
    )


# speedup vs baseline: 1.0424x; 1.0424x over previous
import functools

import jax
import jax.numpy as jnp
from jax import lax
from jax.experimental import pallas as pl
from jax.experimental.pallas import tpu as pltpu
from jax.experimental.pallas import tpu_sc as plsc

BINS = 100000
EMB = 16
NF = 26

NC = 2
NS = 16
LANES = 16
NW = NC * NS

CHUNK = 128


def _sc_gather_call(tot):
    bpw = tot // NW
    nchunk = NF
    bt = bpw // NF

    mesh = plsc.VectorSubcoreMesh(core_axis_name="c", subcore_axis_name="s")

    @functools.partial(
        pl.kernel,
        out_type=jax.ShapeDtypeStruct((tot * EMB,), jnp.float32),
        mesh=mesh,
        scratch_types=[
            pltpu.VMEM((bpw,), jnp.int32),
            pltpu.VMEM((bpw,), jnp.int32),
            pltpu.VMEM((CHUNK, EMB), jnp.float32),
            pltpu.VMEM((CHUNK, EMB), jnp.float32),
            pltpu.VMEM((bpw * EMB,), jnp.float32),
            pltpu.SemaphoreType.DMA,
            pltpu.SemaphoreType.DMA,
        ],
        compiler_params=pltpu.CompilerParams(
            use_tc_tiling_on_sc=False, needs_layout_passes=False),
    )
    def sc_gather(idx_hbm, tbl_hbm, out_hbm, idx_v, bidx_v, buf_a, buf_b,
                  rows_v, sem_a, sem_b):
        wid = lax.axis_index("s") * NC + lax.axis_index("c")
        base = wid * bpw
        pltpu.sync_copy(idx_hbm.at[pl.ds(base, bpw)], idx_v)

        lane = lax.broadcasted_iota(jnp.int32, (LANES,), 0)

        def xform(f, carry):
            for q in range(bt // LANES):
                bl = q * LANES + lane
                raw = plsc.load_gather(idx_v, [bl * NF + f])
                bidx_v[pl.ds(f * CHUNK + q * LANES, LANES)] = lax.rem(raw, BINS)
            return carry

        lax.fori_loop(0, NF, xform, 0)

        bufs = (buf_a, buf_b)
        sems = (sem_a, sem_b)

        def fire(f, buf, sem):
            pltpu.make_async_copy(
                tbl_hbm.at[f].at[bidx_v.at[pl.ds(f * CHUNK, CHUNK)]], buf, sem,
            ).start()

        def repack(f, buf):
            for q in range(bt // LANES):
                bl = q * LANES + lane
                posb = bl * (NF * EMB) + f * EMB
                rowv = q * LANES + lane
                for p in range(EMB):
                    colv = jnp.full((LANES,), p, jnp.int32)
                    val = plsc.load_gather(buf, [rowv, colv])
                    plsc.store_scatter(rows_v, [posb + p], val)

        fire(0, bufs[0], sems[0])

        def pairbody(g, carry):
            for b in range(2):
                f = g * 2 + b

                @pl.when(f + 1 < nchunk)
                def _():
                    fire(f + 1, bufs[1 - b], sems[1 - b])

                pltpu.make_async_copy(
                    tbl_hbm.at[0].at[pl.ds(0, CHUNK)], bufs[b], sems[b]).wait()
                repack(f, bufs[b])
            return carry

        lax.fori_loop(0, nchunk // 2, pairbody, 0)
        pltpu.sync_copy(rows_v, out_hbm.at[pl.ds(base * EMB, bpw * EMB)])

    return sc_gather


def _mlp_body(dense_ref, embs_ref, w1a, w1b, b1, w2, b2, w3, b3, w4, b4, out_ref):
    f32 = jnp.float32
    h = jnp.dot(embs_ref[...], w1b[...], preferred_element_type=f32)
    h += jnp.dot(dense_ref[...], w1a[...], preferred_element_type=f32)
    h = jnp.maximum(h + b1[...], 0.0)
    h = jnp.maximum(jnp.dot(h, w2[...], preferred_element_type=f32) + b2[...], 0.0)
    h = jnp.maximum(jnp.dot(h, w3[...], preferred_element_type=f32) + b3[...], 0.0)
    o = jnp.dot(h, w4[...], preferred_element_type=f32) + b4[...]
    out_ref[...] = 1.0 / (1.0 + jnp.exp(-o))


def _mlp_call(dense, embs, w1a, w1b, b1, w2, b2, w3, b3, w4, b4, bt=512):
    bsz, nd = dense.shape
    demb = embs.shape[1]
    u1, u2, u3 = w2.shape[0], w3.shape[0], w4.shape[0]
    grid = (bsz // bt,)
    full = lambda shape: pl.BlockSpec(shape, lambda i: (0, 0))
    return pl.pallas_call(
        _mlp_body,
        grid=grid,
        in_specs=[
            pl.BlockSpec((bt, nd), lambda i: (i, 0)),
            pl.BlockSpec((bt, demb), lambda i: (i, 0)),
            full((nd, u1)),
            full((demb, u1)),
            full((1, u1)),
            full((u1, u2)),
            full((1, u2)),
            full((u2, u3)),
            full((1, u3)),
            full((u3, 1)),
            full((1, 1)),
        ],
        out_specs=pl.BlockSpec((bt, 1), lambda i: (i, 0)),
        out_shape=jax.ShapeDtypeStruct((bsz, 1), jnp.float32),
    )(dense, embs, w1a, w1b, b1, w2, b2, w3, b3, w4, b4)


def kernel(dense, sparse_idx, emb_table, W1, b1, W2, b2, W3, b3, W4, b4):
    bsz, nd = dense.shape
    nf, nbins, emb = emb_table.shape
    tot = bsz * nf

    idx_flat = sparse_idx.reshape(tot).astype(jnp.int32)
    rows = _sc_gather_call(tot)(idx_flat, emb_table)
    embs = rows.reshape(bsz, nf * emb)
    return _mlp_call(
        dense, embs,
        W1[:nd], W1[nd:], b1.reshape(1, -1),
        W2, b2.reshape(1, -1),
        W3, b3.reshape(1, -1),
        W4, b4.reshape(1, -1),
    )


# speedup vs baseline: 2.1186x; 2.0324x over previous
import functools

import jax
import jax.numpy as jnp
from jax import lax
from jax.experimental import pallas as pl
from jax.experimental.pallas import tpu as pltpu
from jax.experimental.pallas import tpu_sc as plsc

BINS = 100000
EMB = 16
NF = 26

NC = 2
NS = 16
LANES = 16
NW = NC * NS

CHUNK = 128
BLK = 128
ROWS_PER_SUPER = 8


def _sc_transpose_call():
    nblk_f = (BINS + BLK - 1) // BLK
    nblk = NF * nblk_f
    per_tile = (nblk + NW - 1) // NW
    srows_f = BINS // ROWS_PER_SUPER

    mesh = plsc.VectorSubcoreMesh(core_axis_name="c", subcore_axis_name="s")

    @functools.partial(
        pl.kernel,
        out_type=jax.ShapeDtypeStruct((NF, srows_f, ROWS_PER_SUPER * EMB),
                                      jnp.float32),
        mesh=mesh,
        scratch_types=[
            pltpu.VMEM((EMB, BLK), jnp.float32),
            pltpu.VMEM((EMB, BLK), jnp.float32),
            pltpu.VMEM((BLK // ROWS_PER_SUPER, BLK), jnp.float32),
            pltpu.SemaphoreType.DMA,
            pltpu.SemaphoreType.DMA,
            pltpu.SemaphoreType.DMA,
        ],
        compiler_params=pltpu.CompilerParams(needs_layout_passes=False),
    )
    def sc_transpose(tbl_hbm, out_hbm, buf_a, buf_b, tbuf, sem_a, sem_b, sem_w):
        wid = lax.axis_index("s") * NC + lax.axis_index("c")
        blk0 = wid * per_tile

        lane = lax.broadcasted_iota(jnp.int32, (LANES,), 0)
        bufs = (buf_a, buf_b)
        sems = (sem_a, sem_b)
        tail = BINS % BLK
        sup_blk = BLK // ROWS_PER_SUPER

        def fire(blk, buf, sem):
            f = blk // nblk_f
            c = blk % nblk_f
            cb = pl.multiple_of(c * BLK, BLK)
            pltpu.make_async_copy(
                tbl_hbm.at[f].at[:, pl.ds(cb, BLK)], buf, sem,
            ).start()

        def handle(blk, buf):
            f = blk // nblk_f
            c = blk % nblk_f
            for q in range(BLK // LANES):
                posb = (q * LANES + lane) * EMB
                for e in range(EMB):
                    pos = posb + e
                    val = buf[e, pl.ds(q * LANES, LANES)]
                    plsc.store_scatter(
                        tbuf, [lax.shift_right_logical(pos, 7), pos & 127], val)
            srow = pl.multiple_of(c * sup_blk, sup_blk)

            @pl.when(c < nblk_f - 1)
            def _():
                pltpu.sync_copy(tbuf, out_hbm.at[f].at[pl.ds(srow, sup_blk)])

            @pl.when(c == nblk_f - 1)
            def _():
                pltpu.sync_copy(
                    tbuf.at[pl.ds(0, tail // ROWS_PER_SUPER)],
                    out_hbm.at[f].at[pl.ds(srow, tail // ROWS_PER_SUPER)])

        fire(blk0, bufs[0], sems[0])

        def pairbody(g, carry):
            for b in range(2):
                i = g * 2 + b
                blk = blk0 + i

                @pl.when((i + 1 < per_tile) & (blk + 1 < nblk))
                def _():
                    fire(blk + 1, bufs[1 - b], sems[1 - b])

                @pl.when(blk < nblk)
                def _():
                    pltpu.make_async_copy(
                        tbl_hbm.at[0].at[:, pl.ds(0, BLK)], bufs[b],
                        sems[b]).wait()
                    handle(blk, bufs[b])
            return carry

        lax.fori_loop(0, (per_tile + 1) // 2, pairbody, 0)

    return sc_transpose


def _sc_gather_call(tot):
    bpw = tot // NW
    nchunk = NF
    bt = bpw // NF

    mesh = plsc.VectorSubcoreMesh(core_axis_name="c", subcore_axis_name="s")

    @functools.partial(
        pl.kernel,
        out_type=jax.ShapeDtypeStruct((tot * EMB,), jnp.float32),
        mesh=mesh,
        scratch_types=[
            pltpu.VMEM((bpw,), jnp.int32),
            pltpu.VMEM((bpw,), jnp.int32),
            pltpu.VMEM((bpw,), jnp.int32),
            pltpu.VMEM((CHUNK, 128), jnp.float32),
            pltpu.VMEM((CHUNK, 128), jnp.float32),
            pltpu.VMEM((bpw * EMB,), jnp.float32),
            pltpu.SemaphoreType.DMA,
            pltpu.SemaphoreType.DMA,
        ],
        compiler_params=pltpu.CompilerParams(needs_layout_passes=False),
    )
    def sc_gather(idx_hbm, tbl_hbm, out_hbm, idx_v, sidx_v, soff_v, buf_a,
                  buf_b, rows_v, sem_a, sem_b):
        wid = lax.axis_index("s") * NC + lax.axis_index("c")
        base = wid * bpw
        pltpu.sync_copy(idx_hbm.at[pl.ds(base, bpw)], idx_v)

        lane = lax.broadcasted_iota(jnp.int32, (LANES,), 0)
        srows_f = BINS // ROWS_PER_SUPER

        def xform(f, carry):
            for q in range(bt // LANES):
                bl = q * LANES + lane
                raw = plsc.load_gather(idx_v, [bl * NF + f])
                binned = lax.rem(raw, BINS)
                dst = pl.ds(f * CHUNK + q * LANES, LANES)
                sidx_v[dst] = lax.shift_right_logical(binned, 3)
                soff_v[dst] = (binned & 7) * EMB
            return carry

        lax.fori_loop(0, NF, xform, 0)

        bufs = (buf_a, buf_b)
        sems = (sem_a, sem_b)

        def fire(f, buf, sem):
            pltpu.make_async_copy(
                tbl_hbm.at[f].at[sidx_v.at[pl.ds(f * CHUNK, CHUNK)]], buf, sem,
            ).start()

        def extract(f, buf):
            for q in range(bt // LANES):
                bl = q * LANES + lane
                rowv = q * LANES + lane
                colb = soff_v[pl.ds(f * CHUNK + q * LANES, LANES)]
                posb = bl * (NF * EMB) + f * EMB
                for p in range(EMB):
                    val = plsc.load_gather(buf, [rowv, colb + p])
                    plsc.store_scatter(rows_v, [posb + p], val)

        fire(0, bufs[0], sems[0])

        def pairbody(g, carry):
            for b in range(2):
                f = g * 2 + b

                @pl.when(f + 1 < nchunk)
                def _():
                    fire(f + 1, bufs[1 - b], sems[1 - b])

                pltpu.make_async_copy(
                    tbl_hbm.at[0].at[pl.ds(0, CHUNK)], bufs[b], sems[b]).wait()
                extract(f, bufs[b])
            return carry

        lax.fori_loop(0, nchunk // 2, pairbody, 0)
        pltpu.sync_copy(rows_v, out_hbm.at[pl.ds(base * EMB, bpw * EMB)])

    return sc_gather


def _mlp_body(dense_ref, embs_ref, w1a, w1b, b1, w2, b2, w3, b3, w4, b4, out_ref):
    f32 = jnp.float32
    h = jnp.dot(embs_ref[...], w1b[...], preferred_element_type=f32)
    h += jnp.dot(dense_ref[...], w1a[...], preferred_element_type=f32)
    h = jnp.maximum(h + b1[...], 0.0)
    h = jnp.maximum(jnp.dot(h, w2[...], preferred_element_type=f32) + b2[...], 0.0)
    h = jnp.maximum(jnp.dot(h, w3[...], preferred_element_type=f32) + b3[...], 0.0)
    o = jnp.dot(h, w4[...], preferred_element_type=f32) + b4[...]
    out_ref[...] = 1.0 / (1.0 + jnp.exp(-o))


def _mlp_call(dense, embs, w1a, w1b, b1, w2, b2, w3, b3, w4, b4, bt=512):
    bsz, nd = dense.shape
    demb = embs.shape[1]
    u1, u2, u3 = w2.shape[0], w3.shape[0], w4.shape[0]
    grid = (bsz // bt,)
    full = lambda shape: pl.BlockSpec(shape, lambda i: (0, 0))
    return pl.pallas_call(
        _mlp_body,
        grid=grid,
        in_specs=[
            pl.BlockSpec((bt, nd), lambda i: (i, 0)),
            pl.BlockSpec((bt, demb), lambda i: (i, 0)),
            full((nd, u1)),
            full((demb, u1)),
            full((1, u1)),
            full((u1, u2)),
            full((1, u2)),
            full((u2, u3)),
            full((1, u3)),
            full((u3, 1)),
            full((1, 1)),
        ],
        out_specs=pl.BlockSpec((bt, 1), lambda i: (i, 0)),
        out_shape=jax.ShapeDtypeStruct((bsz, 1), jnp.float32),
    )(dense, embs, w1a, w1b, b1, w2, b2, w3, b3, w4, b4)


def kernel(dense, sparse_idx, emb_table, W1, b1, W2, b2, W3, b3, W4, b4):
    bsz, nd = dense.shape
    nf, nbins, emb = emb_table.shape
    tot = bsz * nf

    idx_flat = sparse_idx.reshape(tot).astype(jnp.int32)
    tblT = emb_table.transpose(0, 2, 1)
    tbl8 = _sc_transpose_call()(tblT)
    rows = _sc_gather_call(tot)(idx_flat, tbl8)
    embs = rows.reshape(bsz, nf * emb)
    return _mlp_call(
        dense, embs,
        W1[:nd], W1[nd:], b1.reshape(1, -1),
        W2, b2.reshape(1, -1),
        W3, b3.reshape(1, -1),
        W4, b4.reshape(1, -1),
    )


# speedup vs baseline: 2.4778x; 1.1695x over previous
import functools

import jax
import jax.numpy as jnp
from jax import lax
from jax.experimental import pallas as pl
from jax.experimental.pallas import tpu as pltpu
from jax.experimental.pallas import tpu_sc as plsc

BINS = 100000
EMB = 16
NF = 26

NC = 2
NS = 16
LANES = 16
NW = NC * NS

CHUNK = 128
BLK = 128
ROWS_PER_SUPER = 8


def _sc_transpose_call():
    blk = 256
    nblk_f = (BINS + blk - 1) // blk
    nblk = NF * nblk_f
    per_tile = (nblk + NW - 1) // NW
    srows_f = BINS // ROWS_PER_SUPER

    mesh = plsc.VectorSubcoreMesh(core_axis_name="c", subcore_axis_name="s")

    @functools.partial(
        pl.kernel,
        out_type=jax.ShapeDtypeStruct((NF, srows_f, ROWS_PER_SUPER * EMB),
                                      jnp.float32),
        mesh=mesh,
        scratch_types=[
            pltpu.VMEM((EMB, blk), jnp.float32),
            pltpu.VMEM((EMB, blk), jnp.float32),
            pltpu.VMEM((blk // ROWS_PER_SUPER, 128), jnp.float32),
            pltpu.SemaphoreType.DMA,
            pltpu.SemaphoreType.DMA,
            pltpu.SemaphoreType.DMA,
        ],
        compiler_params=pltpu.CompilerParams(needs_layout_passes=False),
    )
    def sc_transpose(tbl_hbm, out_hbm, buf_a, buf_b, tbuf, sem_a, sem_b, sem_w):
        wid = lax.axis_index("s") * NC + lax.axis_index("c")
        blk0 = wid * per_tile

        lane = lax.broadcasted_iota(jnp.int32, (LANES,), 0)
        bufs = (buf_a, buf_b)
        sems = (sem_a, sem_b)
        tail = BINS % blk
        sup_blk = blk // ROWS_PER_SUPER
        lane_hi = lax.shift_right_logical(lane, 3)
        lane_lo16 = (lane & 7) * EMB

        def fire(bk, buf, sem):
            f = bk // nblk_f
            c = bk % nblk_f
            cb = pl.multiple_of(c * blk, blk)
            pltpu.make_async_copy(
                tbl_hbm.at[f].at[:, pl.ds(cb, blk)], buf, sem,
            ).start()

        def handle(bk, buf):
            f = bk // nblk_f
            c = bk % nblk_f
            for q in range(blk // LANES):
                rowv = q * 2 + lane_hi
                for e in range(EMB):
                    val = buf[e, pl.ds(q * LANES, LANES)]
                    plsc.store_scatter(tbuf, [rowv, lane_lo16 + e], val)
            srow = pl.multiple_of(c * sup_blk, sup_blk)

            @pl.when(c < nblk_f - 1)
            def _():
                pltpu.sync_copy(tbuf, out_hbm.at[f].at[pl.ds(srow, sup_blk)])

            @pl.when(c == nblk_f - 1)
            def _():
                pltpu.sync_copy(
                    tbuf.at[pl.ds(0, 16)],
                    out_hbm.at[f].at[pl.ds(srow, 16)])
                pltpu.sync_copy(
                    tbuf.at[pl.ds(16, 4)],
                    out_hbm.at[f].at[pl.ds(srow + 16, 4)])

        fire(blk0, bufs[0], sems[0])

        def pairbody(g, carry):
            for b in range(2):
                i = g * 2 + b
                bk = blk0 + i

                @pl.when((i + 1 < per_tile) & (bk + 1 < nblk))
                def _():
                    fire(bk + 1, bufs[1 - b], sems[1 - b])

                @pl.when(bk < nblk)
                def _():
                    pltpu.make_async_copy(
                        tbl_hbm.at[0].at[:, pl.ds(0, blk)], bufs[b],
                        sems[b]).wait()
                    handle(bk, bufs[b])
            return carry

        lax.fori_loop(0, (per_tile + 1) // 2, pairbody, 0)

    return sc_transpose


def _sc_gather_call(tot):
    bpw = tot // NW
    nchunk = NF
    bt = bpw // NF

    mesh = plsc.VectorSubcoreMesh(core_axis_name="c", subcore_axis_name="s")

    @functools.partial(
        pl.kernel,
        out_type=jax.ShapeDtypeStruct((tot * EMB,), jnp.float32),
        mesh=mesh,
        scratch_types=[
            pltpu.VMEM((bpw,), jnp.int32),
            pltpu.VMEM((bpw,), jnp.int32),
            pltpu.VMEM((bpw,), jnp.int32),
            pltpu.VMEM((CHUNK, 128), jnp.float32),
            pltpu.VMEM((CHUNK, 128), jnp.float32),
            pltpu.VMEM((bpw * EMB,), jnp.float32),
            pltpu.SemaphoreType.DMA,
            pltpu.SemaphoreType.DMA,
        ],
        compiler_params=pltpu.CompilerParams(needs_layout_passes=False),
    )
    def sc_gather(idx_hbm, tbl_hbm, out_hbm, idx_v, sidx_v, soff_v, buf_a,
                  buf_b, rows_v, sem_a, sem_b):
        wid = lax.axis_index("s") * NC + lax.axis_index("c")
        base = wid * bpw
        pltpu.sync_copy(idx_hbm.at[pl.ds(base, bpw)], idx_v)

        lane = lax.broadcasted_iota(jnp.int32, (LANES,), 0)
        srows_f = BINS // ROWS_PER_SUPER

        def xform(f, carry):
            for q in range(bt // LANES):
                bl = q * LANES + lane
                raw = plsc.load_gather(idx_v, [bl * NF + f])
                binned = lax.rem(raw, BINS)
                dst = pl.ds(f * CHUNK + q * LANES, LANES)
                sidx_v[dst] = lax.shift_right_logical(binned, 3)
                soff_v[dst] = (binned & 7) * EMB
            return carry

        lax.fori_loop(0, NF, xform, 0)

        bufs = (buf_a, buf_b)
        sems = (sem_a, sem_b)

        def fire(f, buf, sem):
            pltpu.make_async_copy(
                tbl_hbm.at[f].at[sidx_v.at[pl.ds(f * CHUNK, CHUNK)]], buf, sem,
            ).start()

        def extract(f, buf):
            for q in range(bt // LANES):
                bl = q * LANES + lane
                rowv = q * LANES + lane
                colb = soff_v[pl.ds(f * CHUNK + q * LANES, LANES)]
                posb = bl * (NF * EMB) + f * EMB
                for p in range(EMB):
                    val = plsc.load_gather(buf, [rowv, colb + p])
                    plsc.store_scatter(rows_v, [posb + p], val)

        fire(0, bufs[0], sems[0])

        def pairbody(g, carry):
            for b in range(2):
                f = g * 2 + b

                @pl.when(f + 1 < nchunk)
                def _():
                    fire(f + 1, bufs[1 - b], sems[1 - b])

                pltpu.make_async_copy(
                    tbl_hbm.at[0].at[pl.ds(0, CHUNK)], bufs[b], sems[b]).wait()
                extract(f, bufs[b])
            return carry

        lax.fori_loop(0, nchunk // 2, pairbody, 0)
        pltpu.sync_copy(rows_v, out_hbm.at[pl.ds(base * EMB, bpw * EMB)])

    return sc_gather


def _mlp_body(dense_ref, embs_ref, w1a, w1b, b1, w2, b2, w3, b3, w4, b4, out_ref):
    f32 = jnp.float32
    h = jnp.dot(embs_ref[...], w1b[...], preferred_element_type=f32)
    h += jnp.dot(dense_ref[...], w1a[...], preferred_element_type=f32)
    h = jnp.maximum(h + b1[...], 0.0)
    h = jnp.maximum(jnp.dot(h, w2[...], preferred_element_type=f32) + b2[...], 0.0)
    h = jnp.maximum(jnp.dot(h, w3[...], preferred_element_type=f32) + b3[...], 0.0)
    o = jnp.dot(h, w4[...], preferred_element_type=f32) + b4[...]
    out_ref[...] = 1.0 / (1.0 + jnp.exp(-o))


def _mlp_call(dense, embs, w1a, w1b, b1, w2, b2, w3, b3, w4, b4, bt=512):
    bsz, nd = dense.shape
    demb = embs.shape[1]
    u1, u2, u3 = w2.shape[0], w3.shape[0], w4.shape[0]
    grid = (bsz // bt,)
    full = lambda shape: pl.BlockSpec(shape, lambda i: (0, 0))
    return pl.pallas_call(
        _mlp_body,
        grid=grid,
        in_specs=[
            pl.BlockSpec((bt, nd), lambda i: (i, 0)),
            pl.BlockSpec((bt, demb), lambda i: (i, 0)),
            full((nd, u1)),
            full((demb, u1)),
            full((1, u1)),
            full((u1, u2)),
            full((1, u2)),
            full((u2, u3)),
            full((1, u3)),
            full((u3, 1)),
            full((1, 1)),
        ],
        out_specs=pl.BlockSpec((bt, 1), lambda i: (i, 0)),
        out_shape=jax.ShapeDtypeStruct((bsz, 1), jnp.float32),
    )(dense, embs, w1a, w1b, b1, w2, b2, w3, b3, w4, b4)


def kernel(dense, sparse_idx, emb_table, W1, b1, W2, b2, W3, b3, W4, b4):
    bsz, nd = dense.shape
    nf, nbins, emb = emb_table.shape
    tot = bsz * nf

    idx_flat = sparse_idx.reshape(tot).astype(jnp.int32)
    tblT = emb_table.transpose(0, 2, 1)
    tbl8 = _sc_transpose_call()(tblT)
    rows = _sc_gather_call(tot)(idx_flat, tbl8)
    embs = rows.reshape(bsz, nf * emb)
    return _mlp_call(
        dense, embs,
        W1[:nd], W1[nd:], b1.reshape(1, -1),
        W2, b2.reshape(1, -1),
        W3, b3.reshape(1, -1),
        W4, b4.reshape(1, -1),
    )
